# baseline (speedup 1.0000x reference)
import functools

import jax
import jax.numpy as jnp
from jax.experimental import pallas as pl
from jax.experimental.pallas import tpu as pltpu

_RHS_T = (((1,), (1,)), ((), ()))


def _se_step(hw_inv, cr, c, x_ref, p_ref, o_ref):
    w1 = p_ref[0:cr, :]
    w2 = p_ref[cr:cr + c, 0:cr]
    b2 = p_ref[cr + c:cr + c + 1, :]
    b1 = p_ref[cr + c + 1:cr + c + 2, 0:cr]

    xb = x_ref[...].astype(jnp.float32)

    pooled = jnp.sum(xb, axis=2) * hw_inv

    h = jax.lax.dot_general(pooled, w1, _RHS_T,
                            preferred_element_type=jnp.float32)
    h = jnp.maximum(h + b1, 0.0)
    g = jax.lax.dot_general(h, w2, _RHS_T,
                            preferred_element_type=jnp.float32)
    g = jax.nn.sigmoid(g + b2)

    o_ref[...] = (xb * g[:, :, None]).astype(o_ref.dtype)


def _block_images(n, c, hw, itemsize):
    budget = 44 << 20
    lanes = -(-hw // 128) * 128
    per_image = c * lanes * itemsize
    best = 1
    for d in range(1, n + 1):
        if n % d:
            continue
        if 4 * d * per_image <= budget and n // d >= 8:
            best = d
    return best


def kernel(x, fc1_w, fc1_b, fc2_w, fc2_b):
    N, C, H, W = x.shape
    Cr = fc1_w.shape[0]
    HW = H * W

    x_r = x.reshape(N, C, HW)

    rows = Cr + C + 8
    p = jnp.zeros((rows, C), jnp.float32)
    p = jax.lax.dynamic_update_slice(p, fc1_w.astype(jnp.float32), (0, 0))
    p = jax.lax.dynamic_update_slice(p, fc2_w.astype(jnp.float32), (Cr, 0))
    p = jax.lax.dynamic_update_slice(p, fc2_b.reshape(1, C).astype(jnp.float32),
                                     (Cr + C, 0))
    p = jax.lax.dynamic_update_slice(p, fc1_b.reshape(1, Cr).astype(jnp.float32),
                                     (Cr + C + 1, 0))

    nb = _block_images(N, C, HW, x.dtype.itemsize)
    body = functools.partial(_se_step, float(1.0 / HW), Cr, C)

    out_r = pl.pallas_call(
        body,
        out_shape=jax.ShapeDtypeStruct((N, C, HW), x.dtype),
        grid=(N // nb,),
        in_specs=[
            pl.BlockSpec((nb, C, HW), lambda n: (n, 0, 0)),
            pl.BlockSpec((rows, C), lambda n: (0, 0)),
        ],
        out_specs=pl.BlockSpec((nb, C, HW), lambda n: (n, 0, 0)),
        compiler_params=pltpu.CompilerParams(
            dimension_semantics=("parallel",),
            vmem_limit_bytes=56 << 20,
        ),
    )(x_r, p)
    return out_r.reshape(N, C, H, W)


# speedup vs baseline: 1.0400x; 1.0400x over previous
import functools

import jax
import jax.numpy as jnp
from jax.experimental import pallas as pl
from jax.experimental.pallas import tpu as pltpu

_RHS_T = (((1,), (1,)), ((), ()))


def _se_step(hw_inv, x_ref, w1_ref, b1_ref, w2_ref, b2_ref, o_ref):
    xb = x_ref[...].astype(jnp.float32)

    pooled = jnp.sum(xb, axis=2) * hw_inv

    h = jax.lax.dot_general(pooled, w1_ref[...], _RHS_T,
                            preferred_element_type=jnp.float32)
    h = jnp.maximum(h + b1_ref[...], 0.0)
    g = jax.lax.dot_general(h, w2_ref[...], _RHS_T,
                            preferred_element_type=jnp.float32)
    g = jax.nn.sigmoid(g + b2_ref[...])

    o_ref[...] = (xb * g[:, :, None]).astype(o_ref.dtype)


def _block_images(n, c, hw, itemsize):
    budget = 58 << 20
    lanes = -(-hw // 128) * 128
    per_image = c * lanes * itemsize
    best = 1
    for d in range(1, n + 1):
        if n % d:
            continue
        if 4 * d * per_image <= budget and n // d >= 4:
            best = d
    return best


def kernel(x, fc1_w, fc1_b, fc2_w, fc2_b):
    N, C, H, W = x.shape
    Cr = fc1_w.shape[0]
    HW = H * W

    x_r = x.reshape(N, C, HW)
    b1 = fc1_b.reshape(1, Cr)
    b2 = fc2_b.reshape(1, C)

    nb = _block_images(N, C, HW, x.dtype.itemsize)
    body = functools.partial(_se_step, float(1.0 / HW))

    out_r = pl.pallas_call(
        body,
        out_shape=jax.ShapeDtypeStruct((N, C, HW), x.dtype),
        grid=(N // nb,),
        in_specs=[
            pl.BlockSpec((nb, C, HW), lambda n: (n, 0, 0)),
            pl.BlockSpec((Cr, C), lambda n: (0, 0)),
            pl.BlockSpec((1, Cr), lambda n: (0, 0)),
            pl.BlockSpec((C, Cr), lambda n: (0, 0)),
            pl.BlockSpec((1, C), lambda n: (0, 0)),
        ],
        out_specs=pl.BlockSpec((nb, C, HW), lambda n: (n, 0, 0)),
        compiler_params=pltpu.CompilerParams(
            dimension_semantics=("parallel",),
            vmem_limit_bytes=60 << 20,
        ),
    )(x_r, fc1_w, b1, fc2_w, b2)
    return out_r.reshape(N, C, H, W)
